# baseline (speedup 1.0000x reference)
import functools

import jax
import jax.numpy as jnp
from jax import lax
from jax.experimental import pallas as pl
from jax.experimental.pallas import tpu as pltpu
from jax.experimental.pallas import tpu_sc as plsc

CODES = 8192
D = 256
BATCH = 16384
COMMIT = 0.25

BM = 512
BN = 1024
GI = BATCH // BM
GJ = CODES // BN

NC = 2
NS = 16
NW = NC * NS
BPW = BATCH // NW
CH = 128


KB = 2048
NKB = CODES // KB
LW = 128
NG = KB // LW
GB = BM // 8
RB = 4


def _tc_body(z_ref, e_ref, idx_ref, loss_ref,
             bestv_ref, bestg_ref, sz_ref, zs_ref, se_ref):
    i = pl.program_id(0)

    @pl.when(i == 0)
    def _build_se():
        eb = e_ref[...]
        se = jnp.sum(eb * eb, axis=1)
        se_ref[...] = jnp.broadcast_to(se[None, None, :], (1, 8, CODES))

    zb = z_ref[...]
    sz = jnp.sum(zb * zb, axis=1, keepdims=True)
    sz_ref[...] = jnp.broadcast_to(sz, (BM, LW)).reshape(GB, 8, LW)
    zs_ref[...] = zb * -2.0

    def _dot(kb):
        eb = e_ref[pl.ds(kb * KB, KB), :]
        return lax.dot_general(zs_ref[...], eb, (((1,), (1,)), ((), ())),
                               preferred_element_type=jnp.float32
                               ).reshape(GB, 8, KB)

    def _epilogue(kb, dot, first):
        for rc in range(GB // RB):
            r0 = rc * RB
            sz_c = sz_ref[r0:r0 + RB]
            rv = None
            for g in range(NG):
                se = se_ref[:, :, pl.ds(kb * KB + g * LW, LW)]
                d = (sz_c + se) + dot[r0:r0 + RB, :, g * LW:(g + 1) * LW]
                gid = kb * NG + g
                if rv is None:
                    rv = d
                    rg = jnp.broadcast_to(
                        jnp.full((1, 1, 1), gid, jnp.int32), (RB, 8, LW))
                else:
                    upd = d < rv
                    rv = jnp.where(upd, d, rv)
                    rg = jnp.where(upd, gid, rg)
            if first:
                bestv_ref[r0:r0 + RB] = rv
                bestg_ref[r0:r0 + RB] = rg
            else:
                bv = bestv_ref[r0:r0 + RB]
                bg = bestg_ref[r0:r0 + RB]
                upd = rv < bv
                bestv_ref[r0:r0 + RB] = jnp.where(upd, rv, bv)
                bestg_ref[r0:r0 + RB] = jnp.where(upd, rg, bg)

    dot_cur = _dot(0)
    for kb in range(NKB):
        dot_nxt = _dot(kb + 1) if kb + 1 < NKB else None
        _epilogue(kb, dot_cur, kb == 0)
        dot_cur = dot_nxt

    bv = bestv_ref[...]
    vmin = jnp.min(bv, axis=2, keepdims=True)
    lane = lax.broadcasted_iota(jnp.int32, (GB, 8, LW), 2)
    col = bestg_ref[...] * LW + lane
    li = jnp.min(jnp.where(bv == vmin, col, CODES),
                 axis=2, keepdims=True)
    idx_ref[...] = li.reshape(BM, 1)
    psum = jnp.sum(vmin)

    @pl.when(i == 0)
    def _():
        loss_ref[0, 0] = psum

    @pl.when(i > 0)
    def _():
        loss_ref[0, 0] += psum


def _tc_argmin(z_flat, e):
    return pl.pallas_call(
        _tc_body,
        grid=(GI,),
        in_specs=[
            pl.BlockSpec((BM, D), lambda i: (i, 0)),
            pl.BlockSpec((CODES, D), lambda i: (0, 0)),
        ],
        out_specs=[
            pl.BlockSpec((BM, 1), lambda i: (i, 0)),
            pl.BlockSpec(memory_space=pltpu.SMEM),
        ],
        out_shape=[
            jax.ShapeDtypeStruct((BATCH, 1), jnp.int32),
            jax.ShapeDtypeStruct((1, 1), jnp.float32),
        ],
        scratch_shapes=[
            pltpu.VMEM((GB, 8, LW), jnp.float32),
            pltpu.VMEM((GB, 8, LW), jnp.int32),
            pltpu.VMEM((GB, 8, LW), jnp.float32),
            pltpu.VMEM((BM, D), jnp.float32),
            pltpu.VMEM((1, 8, CODES), jnp.float32),
        ],
    )(z_flat, e)


def _sc_gather(table, indices):
    mesh = plsc.VectorSubcoreMesh(
        core_axis_name="c", subcore_axis_name="s",
        num_cores=NC, num_subcores=NS)
    nch = BPW // CH

    @functools.partial(
        pl.kernel,
        out_type=jax.ShapeDtypeStruct((BATCH, D), jnp.float32),
        mesh=mesh,
        scratch_types=[
            pltpu.VMEM((BPW,), jnp.int32),
            pltpu.VMEM((CH, D), jnp.float32),
            pltpu.VMEM((CH, D), jnp.float32),
            pltpu.SemaphoreType.DMA,
            pltpu.SemaphoreType.DMA,
            pltpu.SemaphoreType.DMA,
            pltpu.SemaphoreType.DMA,
        ],
    )
    def gather(table_hbm, idx_hbm, out_hbm, idx_v, rows0, rows1,
               g0, g1, s0, s1):
        wid = lax.axis_index("s") * NC + lax.axis_index("c")
        base = wid * BPW
        bufs, gsem, ssem = [rows0, rows1], [g0, g1], [s0, s1]
        pltpu.sync_copy(idx_hbm.at[pl.ds(base, BPW)], idx_v)

        def _start_gather(c):
            return pltpu.async_copy(
                table_hbm.at[idx_v.at[pl.ds(c * CH, CH)]],
                bufs[c % 2], gsem[c % 2])

        gd = [None] * nch
        sd = [None] * nch
        gd[0] = _start_gather(0)
        if nch > 1:
            gd[1] = _start_gather(1)
        for c in range(nch):
            gd[c].wait()
            sd[c] = pltpu.async_copy(
                bufs[c % 2], out_hbm.at[pl.ds(base + c * CH, CH)],
                ssem[c % 2])
            if c + 2 < nch:
                sd[c].wait()
                gd[c + 2] = _start_gather(c + 2)
        for c in range(max(0, nch - 2), nch):
            sd[c].wait()

    return gather(table, indices)


def kernel(z, embedding_weight):
    original_shape = z.shape
    z_flat = z.reshape(-1, D)
    idx2d, loss_sum = _tc_argmin(z_flat, embedding_weight)
    indices = idx2d.reshape(BATCH)
    z_q = _sc_gather(embedding_weight, indices)
    vq_loss = loss_sum[0, 0] * ((1.0 + COMMIT) / float(BATCH * D))
    return (z_q.reshape(original_shape),
            indices.reshape(original_shape[:-1]),
            vq_loss)


# speedup vs baseline: 1.0038x; 1.0038x over previous
import functools

import jax
import jax.numpy as jnp
from jax import lax
from jax.experimental import pallas as pl
from jax.experimental.pallas import tpu as pltpu
from jax.experimental.pallas import tpu_sc as plsc

CODES = 8192
D = 256
BATCH = 16384
COMMIT = 0.25

BM = 512
BN = 1024
GI = BATCH // BM
GJ = CODES // BN

NC = 2
NS = 16
NW = NC * NS
BPW = BATCH // NW
CH = 128


KB = 2048
NKB = CODES // KB
LW = 128
NG = KB // LW
GB = BM // 8
RB = 16


def _tc_body(z_ref, e_ref, idx_ref, loss_ref,
             bestv_ref, bestg_ref, sz_ref, zs_ref, se_ref):
    i = pl.program_id(0)

    @pl.when(i == 0)
    def _build_se():
        eb = e_ref[...]
        se = jnp.sum(eb * eb, axis=1)
        se_ref[...] = jnp.broadcast_to(se[None, None, :], (1, 8, CODES))

    zb = z_ref[...]
    sz = jnp.sum(zb * zb, axis=1, keepdims=True)
    sz_ref[...] = jnp.broadcast_to(sz, (BM, LW)).reshape(GB, 8, LW)
    zs_ref[...] = zb * -2.0

    def _dot(kb):
        eb = e_ref[pl.ds(kb * KB, KB), :]
        return lax.dot_general(zs_ref[...], eb, (((1,), (1,)), ((), ())),
                               preferred_element_type=jnp.float32
                               ).reshape(GB, 8, KB)

    def _epilogue(kb, dot, first):
        for rc in range(GB // RB):
            r0 = rc * RB
            sz_c = sz_ref[r0:r0 + RB]
            rv = None
            for g in range(NG):
                se = se_ref[:, :, pl.ds(kb * KB + g * LW, LW)]
                d = (sz_c + se) + dot[r0:r0 + RB, :, g * LW:(g + 1) * LW]
                gid = kb * NG + g
                if rv is None:
                    rv = d
                    rg = jnp.broadcast_to(
                        jnp.full((1, 1, 1), gid, jnp.int32), (RB, 8, LW))
                else:
                    upd = d < rv
                    rv = jnp.where(upd, d, rv)
                    rg = jnp.where(upd, gid, rg)
            if first:
                bestv_ref[r0:r0 + RB] = rv
                bestg_ref[r0:r0 + RB] = rg
            else:
                bv = bestv_ref[r0:r0 + RB]
                bg = bestg_ref[r0:r0 + RB]
                upd = rv < bv
                bestv_ref[r0:r0 + RB] = jnp.where(upd, rv, bv)
                bestg_ref[r0:r0 + RB] = jnp.where(upd, rg, bg)

    dot_cur = _dot(0)
    for kb in range(NKB):
        dot_nxt = _dot(kb + 1) if kb + 1 < NKB else None
        _epilogue(kb, dot_cur, kb == 0)
        dot_cur = dot_nxt

    bv = bestv_ref[...]
    vmin = jnp.min(bv, axis=2, keepdims=True)
    lane = lax.broadcasted_iota(jnp.int32, (GB, 8, LW), 2)
    col = bestg_ref[...] * LW + lane
    li = jnp.min(jnp.where(bv == vmin, col, CODES),
                 axis=2, keepdims=True)
    idx_ref[...] = li.reshape(BM, 1)
    psum = jnp.sum(vmin)

    @pl.when(i == 0)
    def _():
        loss_ref[0, 0] = psum

    @pl.when(i > 0)
    def _():
        loss_ref[0, 0] += psum


def _tc_argmin(z_flat, e):
    return pl.pallas_call(
        _tc_body,
        grid=(GI,),
        in_specs=[
            pl.BlockSpec((BM, D), lambda i: (i, 0)),
            pl.BlockSpec((CODES, D), lambda i: (0, 0)),
        ],
        out_specs=[
            pl.BlockSpec((BM, 1), lambda i: (i, 0)),
            pl.BlockSpec(memory_space=pltpu.SMEM),
        ],
        out_shape=[
            jax.ShapeDtypeStruct((BATCH, 1), jnp.int32),
            jax.ShapeDtypeStruct((1, 1), jnp.float32),
        ],
        scratch_shapes=[
            pltpu.VMEM((GB, 8, LW), jnp.float32),
            pltpu.VMEM((GB, 8, LW), jnp.int32),
            pltpu.VMEM((GB, 8, LW), jnp.float32),
            pltpu.VMEM((BM, D), jnp.float32),
            pltpu.VMEM((1, 8, CODES), jnp.float32),
        ],
    )(z_flat, e)


def _sc_gather(table, indices):
    mesh = plsc.VectorSubcoreMesh(
        core_axis_name="c", subcore_axis_name="s",
        num_cores=NC, num_subcores=NS)
    nch = BPW // CH

    @functools.partial(
        pl.kernel,
        out_type=jax.ShapeDtypeStruct((BATCH, D), jnp.float32),
        mesh=mesh,
        scratch_types=[
            pltpu.VMEM((BPW,), jnp.int32),
            pltpu.VMEM((CH, D), jnp.float32),
            pltpu.VMEM((CH, D), jnp.float32),
            pltpu.SemaphoreType.DMA,
            pltpu.SemaphoreType.DMA,
            pltpu.SemaphoreType.DMA,
            pltpu.SemaphoreType.DMA,
        ],
    )
    def gather(table_hbm, idx_hbm, out_hbm, idx_v, rows0, rows1,
               g0, g1, s0, s1):
        wid = lax.axis_index("s") * NC + lax.axis_index("c")
        base = wid * BPW
        bufs, gsem, ssem = [rows0, rows1], [g0, g1], [s0, s1]
        pltpu.sync_copy(idx_hbm.at[pl.ds(base, BPW)], idx_v)

        def _start_gather(c):
            return pltpu.async_copy(
                table_hbm.at[idx_v.at[pl.ds(c * CH, CH)]],
                bufs[c % 2], gsem[c % 2])

        gd = [None] * nch
        sd = [None] * nch
        gd[0] = _start_gather(0)
        if nch > 1:
            gd[1] = _start_gather(1)
        for c in range(nch):
            gd[c].wait()
            sd[c] = pltpu.async_copy(
                bufs[c % 2], out_hbm.at[pl.ds(base + c * CH, CH)],
                ssem[c % 2])
            if c + 2 < nch:
                sd[c].wait()
                gd[c + 2] = _start_gather(c + 2)
        for c in range(max(0, nch - 2), nch):
            sd[c].wait()

    return gather(table, indices)


def kernel(z, embedding_weight):
    original_shape = z.shape
    z_flat = z.reshape(-1, D)
    idx2d, loss_sum = _tc_argmin(z_flat, embedding_weight)
    indices = idx2d.reshape(BATCH)
    z_q = _sc_gather(embedding_weight, indices)
    vq_loss = loss_sum[0, 0] * ((1.0 + COMMIT) / float(BATCH * D))
    return (z_q.reshape(original_shape),
            indices.reshape(original_shape[:-1]),
            vq_loss)


# speedup vs baseline: 1.0468x; 1.0428x over previous
import functools

import jax
import jax.numpy as jnp
from jax import lax
from jax.experimental import pallas as pl
from jax.experimental.pallas import tpu as pltpu
from jax.experimental.pallas import tpu_sc as plsc

CODES = 8192
D = 256
BATCH = 16384
COMMIT = 0.25

BM = 1024
BN = 1024
GI = BATCH // BM
GJ = CODES // BN

NC = 2
NS = 16
NW = NC * NS
BPW = BATCH // NW
CH = 128


KB = 2048
NKB = CODES // KB
LW = 128
NG = KB // LW
GB = BM // 8
RB = 8


def _tc_body(z_ref, e_ref, idx_ref, loss_ref,
             bestv_ref, bestg_ref, sz_ref, zs_ref, se_ref):
    i = pl.program_id(0)

    @pl.when(i == 0)
    def _build_se():
        eb = e_ref[...]
        se = jnp.sum(eb * eb, axis=1)
        se_ref[...] = jnp.broadcast_to(se[None, None, :], (1, 8, CODES))

    zb = z_ref[...]
    sz = jnp.sum(zb * zb, axis=1, keepdims=True)
    sz_ref[...] = jnp.broadcast_to(sz, (BM, LW)).reshape(GB, 8, LW)
    zs_ref[...] = zb * -2.0

    def _dot(kb):
        eb = e_ref[pl.ds(kb * KB, KB), :]
        return lax.dot_general(zs_ref[...], eb, (((1,), (1,)), ((), ())),
                               preferred_element_type=jnp.float32
                               ).reshape(GB, 8, KB)

    def _epilogue(kb, dot, first):
        for rc in range(GB // RB):
            r0 = rc * RB
            sz_c = sz_ref[r0:r0 + RB]
            rv = None
            for g in range(NG):
                se = se_ref[:, :, pl.ds(kb * KB + g * LW, LW)]
                d = (sz_c + se) + dot[r0:r0 + RB, :, g * LW:(g + 1) * LW]
                gid = kb * NG + g
                if rv is None:
                    rv = d
                    rg = jnp.broadcast_to(
                        jnp.full((1, 1, 1), gid, jnp.int32), (RB, 8, LW))
                else:
                    upd = d < rv
                    rv = jnp.where(upd, d, rv)
                    rg = jnp.where(upd, gid, rg)
            if first:
                bestv_ref[r0:r0 + RB] = rv
                bestg_ref[r0:r0 + RB] = rg
            else:
                bv = bestv_ref[r0:r0 + RB]
                bg = bestg_ref[r0:r0 + RB]
                upd = rv < bv
                bestv_ref[r0:r0 + RB] = jnp.where(upd, rv, bv)
                bestg_ref[r0:r0 + RB] = jnp.where(upd, rg, bg)

    dot_cur = _dot(0)
    for kb in range(NKB):
        dot_nxt = _dot(kb + 1) if kb + 1 < NKB else None
        _epilogue(kb, dot_cur, kb == 0)
        dot_cur = dot_nxt

    bv = bestv_ref[...]
    vmin = jnp.min(bv, axis=2, keepdims=True)
    lane = lax.broadcasted_iota(jnp.int32, (GB, 8, LW), 2)
    col = bestg_ref[...] * LW + lane
    li = jnp.min(jnp.where(bv == vmin, col, CODES),
                 axis=2, keepdims=True)
    idx_ref[...] = li.reshape(BM, 1)
    psum = jnp.sum(vmin)

    @pl.when(i == 0)
    def _():
        loss_ref[0, 0] = psum

    @pl.when(i > 0)
    def _():
        loss_ref[0, 0] += psum


def _tc_argmin(z_flat, e):
    return pl.pallas_call(
        _tc_body,
        grid=(GI,),
        in_specs=[
            pl.BlockSpec((BM, D), lambda i: (i, 0)),
            pl.BlockSpec((CODES, D), lambda i: (0, 0)),
        ],
        out_specs=[
            pl.BlockSpec((BM, 1), lambda i: (i, 0)),
            pl.BlockSpec(memory_space=pltpu.SMEM),
        ],
        out_shape=[
            jax.ShapeDtypeStruct((BATCH, 1), jnp.int32),
            jax.ShapeDtypeStruct((1, 1), jnp.float32),
        ],
        scratch_shapes=[
            pltpu.VMEM((GB, 8, LW), jnp.float32),
            pltpu.VMEM((GB, 8, LW), jnp.int32),
            pltpu.VMEM((GB, 8, LW), jnp.float32),
            pltpu.VMEM((BM, D), jnp.float32),
            pltpu.VMEM((1, 8, CODES), jnp.float32),
        ],
    )(z_flat, e)


def _sc_gather(table, indices):
    mesh = plsc.VectorSubcoreMesh(
        core_axis_name="c", subcore_axis_name="s",
        num_cores=NC, num_subcores=NS)
    nch = BPW // CH

    @functools.partial(
        pl.kernel,
        out_type=jax.ShapeDtypeStruct((BATCH, D), jnp.float32),
        mesh=mesh,
        scratch_types=[
            pltpu.VMEM((BPW,), jnp.int32),
            pltpu.VMEM((CH, D), jnp.float32),
            pltpu.VMEM((CH, D), jnp.float32),
            pltpu.SemaphoreType.DMA,
            pltpu.SemaphoreType.DMA,
            pltpu.SemaphoreType.DMA,
            pltpu.SemaphoreType.DMA,
        ],
    )
    def gather(table_hbm, idx_hbm, out_hbm, idx_v, rows0, rows1,
               g0, g1, s0, s1):
        wid = lax.axis_index("s") * NC + lax.axis_index("c")
        base = wid * BPW
        bufs, gsem, ssem = [rows0, rows1], [g0, g1], [s0, s1]
        pltpu.sync_copy(idx_hbm.at[pl.ds(base, BPW)], idx_v)

        def _start_gather(c):
            return pltpu.async_copy(
                table_hbm.at[idx_v.at[pl.ds(c * CH, CH)]],
                bufs[c % 2], gsem[c % 2])

        gd = [None] * nch
        sd = [None] * nch
        gd[0] = _start_gather(0)
        if nch > 1:
            gd[1] = _start_gather(1)
        for c in range(nch):
            gd[c].wait()
            sd[c] = pltpu.async_copy(
                bufs[c % 2], out_hbm.at[pl.ds(base + c * CH, CH)],
                ssem[c % 2])
            if c + 2 < nch:
                sd[c].wait()
                gd[c + 2] = _start_gather(c + 2)
        for c in range(max(0, nch - 2), nch):
            sd[c].wait()

    return gather(table, indices)


def kernel(z, embedding_weight):
    original_shape = z.shape
    z_flat = z.reshape(-1, D)
    idx2d, loss_sum = _tc_argmin(z_flat, embedding_weight)
    indices = idx2d.reshape(BATCH)
    z_q = _sc_gather(embedding_weight, indices)
    vq_loss = loss_sum[0, 0] * ((1.0 + COMMIT) / float(BATCH * D))
    return (z_q.reshape(original_shape),
            indices.reshape(original_shape[:-1]),
            vq_loss)


# speedup vs baseline: 1.0516x; 1.0046x over previous
import functools

import jax
import jax.numpy as jnp
from jax import lax
from jax.experimental import pallas as pl
from jax.experimental.pallas import tpu as pltpu
from jax.experimental.pallas import tpu_sc as plsc

CODES = 8192
D = 256
BATCH = 16384
COMMIT = 0.25

BM = 2048
BN = 1024
GI = BATCH // BM
GJ = CODES // BN

NC = 2
NS = 16
NW = NC * NS
BPW = BATCH // NW
CH = 128


KB = 2048
NKB = CODES // KB
LW = 128
NG = KB // LW
GB = BM // 8
RB = 8


def _tc_body(z_ref, e_ref, idx_ref, loss_ref,
             bestv_ref, bestg_ref, sz_ref, zs_ref, se_ref):
    i = pl.program_id(0)

    @pl.when(i == 0)
    def _build_se():
        eb = e_ref[...]
        se = jnp.sum(eb * eb, axis=1)
        se_ref[...] = jnp.broadcast_to(se[None, None, :], (1, 8, CODES))

    zb = z_ref[...]
    sz = jnp.sum(zb * zb, axis=1, keepdims=True)
    sz_ref[...] = jnp.broadcast_to(sz, (BM, LW)).reshape(GB, 8, LW)
    zs_ref[...] = zb * -2.0

    def _dot(kb):
        eb = e_ref[pl.ds(kb * KB, KB), :]
        return lax.dot_general(zs_ref[...], eb, (((1,), (1,)), ((), ())),
                               preferred_element_type=jnp.float32
                               ).reshape(GB, 8, KB)

    def _epilogue(kb, dot, first):
        for rc in range(GB // RB):
            r0 = rc * RB
            sz_c = sz_ref[r0:r0 + RB]
            rv = None
            for g in range(NG):
                se = se_ref[:, :, pl.ds(kb * KB + g * LW, LW)]
                d = (sz_c + se) + dot[r0:r0 + RB, :, g * LW:(g + 1) * LW]
                gid = kb * NG + g
                if rv is None:
                    rv = d
                    rg = jnp.broadcast_to(
                        jnp.full((1, 1, 1), gid, jnp.int32), (RB, 8, LW))
                else:
                    upd = d < rv
                    rv = jnp.where(upd, d, rv)
                    rg = jnp.where(upd, gid, rg)
            if first:
                bestv_ref[r0:r0 + RB] = rv
                bestg_ref[r0:r0 + RB] = rg
            else:
                bv = bestv_ref[r0:r0 + RB]
                bg = bestg_ref[r0:r0 + RB]
                upd = rv < bv
                bestv_ref[r0:r0 + RB] = jnp.where(upd, rv, bv)
                bestg_ref[r0:r0 + RB] = jnp.where(upd, rg, bg)

    dot_cur = _dot(0)
    for kb in range(NKB):
        dot_nxt = _dot(kb + 1) if kb + 1 < NKB else None
        _epilogue(kb, dot_cur, kb == 0)
        dot_cur = dot_nxt

    bv = bestv_ref[...]
    vmin = jnp.min(bv, axis=2, keepdims=True)
    lane = lax.broadcasted_iota(jnp.int32, (GB, 8, LW), 2)
    col = bestg_ref[...] * LW + lane
    li = jnp.min(jnp.where(bv == vmin, col, CODES),
                 axis=2, keepdims=True)
    idx_ref[...] = li.reshape(BM, 1)
    psum = jnp.sum(vmin)

    @pl.when(i == 0)
    def _():
        loss_ref[0, 0] = psum

    @pl.when(i > 0)
    def _():
        loss_ref[0, 0] += psum


def _tc_argmin(z_flat, e):
    return pl.pallas_call(
        _tc_body,
        grid=(GI,),
        in_specs=[
            pl.BlockSpec((BM, D), lambda i: (i, 0)),
            pl.BlockSpec((CODES, D), lambda i: (0, 0)),
        ],
        out_specs=[
            pl.BlockSpec((BM, 1), lambda i: (i, 0)),
            pl.BlockSpec(memory_space=pltpu.SMEM),
        ],
        out_shape=[
            jax.ShapeDtypeStruct((BATCH, 1), jnp.int32),
            jax.ShapeDtypeStruct((1, 1), jnp.float32),
        ],
        scratch_shapes=[
            pltpu.VMEM((GB, 8, LW), jnp.float32),
            pltpu.VMEM((GB, 8, LW), jnp.int32),
            pltpu.VMEM((GB, 8, LW), jnp.float32),
            pltpu.VMEM((BM, D), jnp.float32),
            pltpu.VMEM((1, 8, CODES), jnp.float32),
        ],
    )(z_flat, e)


def _sc_gather(table, indices):
    mesh = plsc.VectorSubcoreMesh(
        core_axis_name="c", subcore_axis_name="s",
        num_cores=NC, num_subcores=NS)
    nch = BPW // CH

    @functools.partial(
        pl.kernel,
        out_type=jax.ShapeDtypeStruct((BATCH, D), jnp.float32),
        mesh=mesh,
        scratch_types=[
            pltpu.VMEM((BPW,), jnp.int32),
            pltpu.VMEM((CH, D), jnp.float32),
            pltpu.VMEM((CH, D), jnp.float32),
            pltpu.SemaphoreType.DMA,
            pltpu.SemaphoreType.DMA,
            pltpu.SemaphoreType.DMA,
            pltpu.SemaphoreType.DMA,
        ],
    )
    def gather(table_hbm, idx_hbm, out_hbm, idx_v, rows0, rows1,
               g0, g1, s0, s1):
        wid = lax.axis_index("s") * NC + lax.axis_index("c")
        base = wid * BPW
        bufs, gsem, ssem = [rows0, rows1], [g0, g1], [s0, s1]
        pltpu.sync_copy(idx_hbm.at[pl.ds(base, BPW)], idx_v)

        def _start_gather(c):
            return pltpu.async_copy(
                table_hbm.at[idx_v.at[pl.ds(c * CH, CH)]],
                bufs[c % 2], gsem[c % 2])

        gd = [None] * nch
        sd = [None] * nch
        gd[0] = _start_gather(0)
        if nch > 1:
            gd[1] = _start_gather(1)
        for c in range(nch):
            gd[c].wait()
            sd[c] = pltpu.async_copy(
                bufs[c % 2], out_hbm.at[pl.ds(base + c * CH, CH)],
                ssem[c % 2])
            if c + 2 < nch:
                sd[c].wait()
                gd[c + 2] = _start_gather(c + 2)
        for c in range(max(0, nch - 2), nch):
            sd[c].wait()

    return gather(table, indices)


def kernel(z, embedding_weight):
    original_shape = z.shape
    z_flat = z.reshape(-1, D)
    idx2d, loss_sum = _tc_argmin(z_flat, embedding_weight)
    indices = idx2d.reshape(BATCH)
    z_q = _sc_gather(embedding_weight, indices)
    vq_loss = loss_sum[0, 0] * ((1.0 + COMMIT) / float(BATCH * D))
    return (z_q.reshape(original_shape),
            indices.reshape(original_shape[:-1]),
            vq_loss)


# speedup vs baseline: 1.0659x; 1.0135x over previous
import functools

import jax
import jax.numpy as jnp
from jax import lax
from jax.experimental import pallas as pl
from jax.experimental.pallas import tpu as pltpu
from jax.experimental.pallas import tpu_sc as plsc

CODES = 8192
D = 256
BATCH = 16384
COMMIT = 0.25

BM = 2048
BN = 1024
GI = BATCH // BM
GJ = CODES // BN

NC = 2
NS = 16
NW = NC * NS
BPW = BATCH // NW
CH = 128


KB = 1024
NKB = CODES // KB
LW = 128
NG = KB // LW
GB = BM // 8
RB = 8


def _tc_body(z_ref, e_ref, idx_ref, loss_ref,
             bestv_ref, bestg_ref, sz_ref, zs_ref, se_ref):
    i = pl.program_id(0)

    @pl.when(i == 0)
    def _build_se():
        eb = e_ref[...]
        se = jnp.sum(eb * eb, axis=1)
        se_ref[...] = jnp.broadcast_to(se[None, None, :], (1, 8, CODES))

    zb = z_ref[...]
    sz = jnp.sum(zb * zb, axis=1, keepdims=True)
    sz_ref[...] = jnp.broadcast_to(sz, (BM, LW)).reshape(GB, 8, LW)
    zs_ref[...] = zb * -2.0

    def _dot(kb):
        eb = e_ref[pl.ds(kb * KB, KB), :]
        return lax.dot_general(zs_ref[...], eb, (((1,), (1,)), ((), ())),
                               preferred_element_type=jnp.float32
                               ).reshape(GB, 8, KB)

    def _epilogue(kb, dot, first):
        for rc in range(GB // RB):
            r0 = rc * RB
            sz_c = sz_ref[r0:r0 + RB]
            rv = None
            for g in range(NG):
                se = se_ref[:, :, pl.ds(kb * KB + g * LW, LW)]
                d = (sz_c + se) + dot[r0:r0 + RB, :, g * LW:(g + 1) * LW]
                gid = kb * NG + g
                if rv is None:
                    rv = d
                    rg = jnp.broadcast_to(
                        jnp.full((1, 1, 1), gid, jnp.int32), (RB, 8, LW))
                else:
                    upd = d < rv
                    rv = jnp.where(upd, d, rv)
                    rg = jnp.where(upd, gid, rg)
            if first:
                bestv_ref[r0:r0 + RB] = rv
                bestg_ref[r0:r0 + RB] = rg
            else:
                bv = bestv_ref[r0:r0 + RB]
                bg = bestg_ref[r0:r0 + RB]
                upd = rv < bv
                bestv_ref[r0:r0 + RB] = jnp.where(upd, rv, bv)
                bestg_ref[r0:r0 + RB] = jnp.where(upd, rg, bg)

    dot_cur = _dot(0)
    for kb in range(NKB):
        dot_nxt = _dot(kb + 1) if kb + 1 < NKB else None
        _epilogue(kb, dot_cur, kb == 0)
        dot_cur = dot_nxt

    bv = bestv_ref[...]
    vmin = jnp.min(bv, axis=2, keepdims=True)
    lane = lax.broadcasted_iota(jnp.int32, (GB, 8, LW), 2)
    col = bestg_ref[...] * LW + lane
    li = jnp.min(jnp.where(bv == vmin, col, CODES),
                 axis=2, keepdims=True)
    idx_ref[...] = li.reshape(BM, 1)
    psum = jnp.sum(vmin)

    @pl.when(i == 0)
    def _():
        loss_ref[0, 0] = psum

    @pl.when(i > 0)
    def _():
        loss_ref[0, 0] += psum


def _tc_argmin(z_flat, e):
    return pl.pallas_call(
        _tc_body,
        grid=(GI,),
        in_specs=[
            pl.BlockSpec((BM, D), lambda i: (i, 0)),
            pl.BlockSpec((CODES, D), lambda i: (0, 0)),
        ],
        out_specs=[
            pl.BlockSpec((BM, 1), lambda i: (i, 0)),
            pl.BlockSpec(memory_space=pltpu.SMEM),
        ],
        out_shape=[
            jax.ShapeDtypeStruct((BATCH, 1), jnp.int32),
            jax.ShapeDtypeStruct((1, 1), jnp.float32),
        ],
        scratch_shapes=[
            pltpu.VMEM((GB, 8, LW), jnp.float32),
            pltpu.VMEM((GB, 8, LW), jnp.int32),
            pltpu.VMEM((GB, 8, LW), jnp.float32),
            pltpu.VMEM((BM, D), jnp.float32),
            pltpu.VMEM((1, 8, CODES), jnp.float32),
        ],
    )(z_flat, e)


def _sc_gather(table, indices):
    mesh = plsc.VectorSubcoreMesh(
        core_axis_name="c", subcore_axis_name="s",
        num_cores=NC, num_subcores=NS)
    nch = BPW // CH

    @functools.partial(
        pl.kernel,
        out_type=jax.ShapeDtypeStruct((BATCH, D), jnp.float32),
        mesh=mesh,
        scratch_types=[
            pltpu.VMEM((BPW,), jnp.int32),
            pltpu.VMEM((CH, D), jnp.float32),
            pltpu.VMEM((CH, D), jnp.float32),
            pltpu.SemaphoreType.DMA,
            pltpu.SemaphoreType.DMA,
            pltpu.SemaphoreType.DMA,
            pltpu.SemaphoreType.DMA,
        ],
    )
    def gather(table_hbm, idx_hbm, out_hbm, idx_v, rows0, rows1,
               g0, g1, s0, s1):
        wid = lax.axis_index("s") * NC + lax.axis_index("c")
        base = wid * BPW
        bufs, gsem, ssem = [rows0, rows1], [g0, g1], [s0, s1]
        pltpu.sync_copy(idx_hbm.at[pl.ds(base, BPW)], idx_v)

        def _start_gather(c):
            return pltpu.async_copy(
                table_hbm.at[idx_v.at[pl.ds(c * CH, CH)]],
                bufs[c % 2], gsem[c % 2])

        gd = [None] * nch
        sd = [None] * nch
        gd[0] = _start_gather(0)
        if nch > 1:
            gd[1] = _start_gather(1)
        for c in range(nch):
            gd[c].wait()
            sd[c] = pltpu.async_copy(
                bufs[c % 2], out_hbm.at[pl.ds(base + c * CH, CH)],
                ssem[c % 2])
            if c + 2 < nch:
                sd[c].wait()
                gd[c + 2] = _start_gather(c + 2)
        for c in range(max(0, nch - 2), nch):
            sd[c].wait()

    return gather(table, indices)


def kernel(z, embedding_weight):
    original_shape = z.shape
    z_flat = z.reshape(-1, D)
    idx2d, loss_sum = _tc_argmin(z_flat, embedding_weight)
    indices = idx2d.reshape(BATCH)
    z_q = _sc_gather(embedding_weight, indices)
    vq_loss = loss_sum[0, 0] * ((1.0 + COMMIT) / float(BATCH * D))
    return (z_q.reshape(original_shape),
            indices.reshape(original_shape[:-1]),
            vq_loss)
